# Initial kernel scaffold; baseline (speedup 1.0000x reference)
import jax
import jax.numpy as jnp
from jax.experimental import pallas as pl


def kernel(embeddings, batch_id):
    raise NotImplementedError("write your pallas kernel here")


# speedup vs baseline: 5.7191x; 5.7191x over previous
import functools

import jax
import jax.numpy as jnp
from jax.experimental import pallas as pl

K = 64
NITER = 25
CHUNK = 4096
HIGHEST = jax.lax.Precision.HIGHEST


def _dot(a, b, dims, precision=HIGHEST):
    return jax.lax.dot_general(
        a, b, dimension_numbers=(dims, ((), ())),
        precision=precision, preferred_element_type=jnp.float32)


def _neg_sampler_kernel(emb_ref, out_ref):
    n = emb_ref.shape[0]
    nch = n // CHUNK
    iota_k = jax.lax.broadcasted_iota(jnp.int32, (1, K), 1)
    ones_col = jnp.ones((CHUNK, 1), jnp.float32)

    def argmin_col(d):
        dmin = jnp.min(d, axis=1, keepdims=True)
        return jnp.min(jnp.where(d == dmin, iota_k, K), axis=1, keepdims=True)

    def argmax_col(d):
        dmax = jnp.max(d, axis=1, keepdims=True)
        return jnp.min(jnp.where(d == dmax, iota_k, K), axis=1, keepdims=True)

    def dist_chunk(j, cent, c2):
        x = emb_ref[pl.ds(j * CHUNK, CHUNK), :]
        x2c = jnp.sum(x * x, axis=1, keepdims=True)
        d = x2c + c2 - 2.0 * _dot(x, cent, ((1,), (1,)),
                                  precision=jax.lax.Precision.DEFAULT)
        return x, d

    def body(_, cent):
        c2 = jnp.sum(cent * cent, axis=1)[None, :]

        def chunk_body(j, acc):
            sums, counts = acc
            x, d = dist_chunk(j, cent, c2)
            onehot = (argmin_col(d) == iota_k).astype(jnp.float32)
            sums = sums + _dot(onehot, x, ((0,), (0,)))
            counts = counts + _dot(onehot, ones_col, ((0,), (0,)))
            return sums, counts

        sums, counts = jax.lax.fori_loop(
            0, nch, chunk_body,
            (jnp.zeros((K, 128), jnp.float32), jnp.zeros((K, 1), jnp.float32)))
        return jnp.where(counts > 0.0, sums / jnp.maximum(counts, 1.0), cent)

    cent = jax.lax.fori_loop(0, NITER, body, emb_ref[:K, :])

    c2 = jnp.sum(cent * cent, axis=1)[None, :]

    def out_body(j, _):
        _, d = dist_chunk(j, cent, c2)
        i1 = argmax_col(d)
        i2 = argmax_col(jnp.where(iota_k == i1, -jnp.inf, d))
        sel = (i2 == iota_k).astype(jnp.float32)
        neg = _dot(sel, cent, ((1,), (0,)))
        out_ref[pl.ds(j * CHUNK, CHUNK), :] = neg
        return 0

    jax.lax.fori_loop(0, nch, out_body, 0)


@functools.partial(jax.jit, static_argnames=())
def kernel(embeddings, batch_id):
    del batch_id
    n, dim = embeddings.shape
    return pl.pallas_call(
        _neg_sampler_kernel,
        out_shape=jax.ShapeDtypeStruct((n, dim), jnp.float32),
        in_specs=[pl.BlockSpec((n, dim), lambda: (0, 0))],
        out_specs=pl.BlockSpec((n, dim), lambda: (0, 0)),
    )(embeddings)
